# Initial kernel scaffold; baseline (speedup 1.0000x reference)
import jax
import jax.numpy as jnp
from jax.experimental import pallas as pl


def kernel(x, edge_index, W, att_w, lin_w, lin_b):
    raise NotImplementedError("write your pallas kernel here")


# speedup vs baseline: 7.9968x; 7.9968x over previous
import functools

import jax
import jax.numpy as jnp
from jax import lax
from jax.experimental import pallas as pl
from jax.experimental.pallas import tpu as pltpu
from jax.experimental.pallas import tpu_sc as plsc

N = 10000
E = 320000
CIN = 128
COUT = 256
NP = 10240
F = 144
NBLK = 2512
EPAD = NBLK * 128
NTILE = 16
TPB = NBLK // NTILE
RB = 2048
GRID = NP // RB
RPT = NP // NTILE


def _h_alpha_body(x_ref, w_ref, aw_ref, hp_ref, alpha_ref, hsum_ref):
    i = pl.program_id(0)
    xb = x_ref[...]
    hb = lax.dot_general(xb, w_ref[...], (((1,), (1,)), ((), ())),
                         preferred_element_type=jnp.float32)
    a = jnp.sum(hb * aw_ref[...], axis=1, keepdims=True)
    s = jnp.sum(hb, axis=1, keepdims=True)
    al = a * s
    alpha_ref[...] = jnp.where(al >= 0, al, 0.2 * al)
    rows = i * RB + lax.broadcasted_iota(jnp.int32, (RB, 1), 0)
    ones = jnp.where(rows < N, 1.0, 0.0).astype(jnp.float32)
    pad = jnp.zeros((RB, F - CIN - 1), jnp.float32)
    hp_ref[0, :, :] = jnp.concatenate([hb[:, :CIN], ones, pad], axis=1)
    hp_ref[1, :, :] = jnp.concatenate([hb[:, CIN:], ones, pad], axis=1)

    @pl.when(i == 0)
    def _():
        hsum_ref[...] = jnp.zeros_like(hsum_ref)

    hsum_ref[...] += jnp.sum(hb, axis=0, keepdims=True)


_h_alpha = pl.pallas_call(
    _h_alpha_body,
    grid=(GRID,),
    in_specs=[
        pl.BlockSpec((RB, CIN), lambda i: (i, 0)),
        pl.BlockSpec((COUT, CIN), lambda i: (0, 0)),
        pl.BlockSpec((1, COUT), lambda i: (0, 0)),
    ],
    out_specs=[
        pl.BlockSpec((2, RB, F), lambda i: (0, i, 0)),
        pl.BlockSpec((RB, 1), lambda i: (i, 0)),
        pl.BlockSpec((1, COUT), lambda i: (0, 0)),
    ],
    out_shape=[
        jax.ShapeDtypeStruct((2, NP, F), jnp.float32),
        jax.ShapeDtypeStruct((NP, 1), jnp.float32),
        jax.ShapeDtypeStruct((1, COUT), jnp.float32),
    ],
)


@functools.partial(
    pl.kernel,
    mesh=plsc.VectorSubcoreMesh(core_axis_name="c", subcore_axis_name="s"),
    compiler_params=pltpu.CompilerParams(use_tc_tiling_on_sc=False),
    out_type=jax.ShapeDtypeStruct((2 * NP, F), jnp.float32),
    scratch_types=[
        pltpu.VMEM((128,), jnp.int32),
        pltpu.VMEM((128,), jnp.int32),
        pltpu.VMEM((128, F), jnp.float32),
        pltpu.VMEM((128, F), jnp.float32),
        pltpu.VMEM_SHARED((NP, F), jnp.float32),
        pltpu.SemaphoreType.DMA,
    ],
)
def _seg_sum(hflat, cgf, rsp, zrows, s_out, idx_v, r_v, rows_v, zbuf, s_acc, sem):
    c = lax.axis_index("c")
    s = lax.axis_index("s")
    pltpu.sync_copy(zrows, zbuf)
    for kk in range(RPT // 128):
        pltpu.sync_copy(zbuf, s_acc.at[pl.ds(s * RPT + kk * 128, 128)])
    plsc.subcore_barrier()

    def body(j, carry):
        base = (s * TPB + j) * 128
        pltpu.sync_copy(cgf.at[pl.ds(c * EPAD + base, 128)], idx_v)
        pltpu.sync_copy(rsp.at[pl.ds(base, 128)], r_v)
        pltpu.async_copy(hflat.at[idx_v], rows_v, sem).wait()
        pltpu.sync_copy(rows_v, s_acc.at[r_v], add=True)
        return carry

    lax.fori_loop(0, TPB, body, 0)
    plsc.subcore_barrier()
    pltpu.sync_copy(s_acc.at[pl.ds(s * RPT, RPT)],
                    s_out.at[pl.ds(c * NP + s * RPT, RPT)])


def _final_body(s0_ref, s1_ref, alpha_ref, hsum_ref, lw_ref, lb_ref, o_ref):
    p0 = s0_ref[...]
    p1 = s1_ref[...]
    S = jnp.concatenate([p0[:, :CIN], p1[:, :CIN]], axis=1)
    d = p0[:, CIN:CIN + 1]
    al = alpha_ref[...]
    t = jnp.exp(-jnp.abs(al))
    pos = al >= 0
    coef_s = jnp.where(pos, 1.0 - t, t - 1.0)
    coef_h = jnp.where(pos, t, 1.0)
    num = coef_s * S + coef_h * hsum_ref[...]
    den = coef_s * d + jnp.where(pos, float(N) * t, float(N))
    out = num / den
    y = lax.dot_general(out, lw_ref[...], (((1,), (1,)), ((), ())),
                        preferred_element_type=jnp.float32) + lb_ref[...]
    o_ref[...] = jnp.where(y > 0, y, jnp.exp(jnp.minimum(y, 0.0)) - 1.0)


_final = pl.pallas_call(
    _final_body,
    grid=(GRID,),
    in_specs=[
        pl.BlockSpec((RB, F), lambda i: (i, 0)),
        pl.BlockSpec((RB, F), lambda i: (i + GRID, 0)),
        pl.BlockSpec((RB, 1), lambda i: (i, 0)),
        pl.BlockSpec((1, COUT), lambda i: (0, 0)),
        pl.BlockSpec((COUT, COUT), lambda i: (0, 0)),
        pl.BlockSpec((1, COUT), lambda i: (0, 0)),
    ],
    out_specs=pl.BlockSpec((RB, COUT), lambda i: (i, 0)),
    out_shape=jax.ShapeDtypeStruct((NP, COUT), jnp.float32),
)


def kernel(x, edge_index, W, att_w, lin_w, lin_b):
    row = edge_index[0].astype(jnp.int32)
    col = edge_index[1].astype(jnp.int32)
    key = row * N + col
    sk = jnp.sort(key)
    uniq = jnp.concatenate([jnp.ones((1,), jnp.bool_), sk[1:] != sk[:-1]])
    r_s = sk // N
    c_s = sk - r_s * N
    spread = jnp.arange(E, dtype=jnp.int32) % 32
    cg = jnp.where(uniq, c_s, N + spread)
    padspread = jnp.arange(EPAD - E, dtype=jnp.int32) % 32
    cgp = jnp.concatenate([cg, N + padspread])
    cgf = jnp.concatenate([cgp, cgp + NP])
    rsp = jnp.concatenate([r_s, N + 100 + padspread]).astype(jnp.int32)

    x_pad = jnp.zeros((NP, CIN), jnp.float32).at[:N].set(x)
    zrows = jnp.zeros((128, F), jnp.float32)

    hp, alpha, hsum = _h_alpha(x_pad, W, att_w)
    hflat = hp.reshape(2 * NP, F)
    s_flat = _seg_sum(hflat, cgf, rsp, zrows)
    out = _final(s_flat, s_flat, alpha, hsum, lin_w,
                 lin_b.reshape(1, COUT))
    return out[:N]
